# baseline (speedup 1.0000x reference)
import functools

import jax
import jax.numpy as jnp
from jax import lax
from jax.experimental import pallas as pl
from jax.experimental.pallas import tpu as pltpu
from jax.experimental.pallas import tpu_sc as plsc

N = 10000
N_PAD = 10240
E = 320000
F_IN = 128
H1 = 128
EMB = 64
ACT = 18

NUM_CORES = 2
NUM_SUBCORES = 16
NW = NUM_CORES * NUM_SUBCORES
CHUNK = 96
SCHUNKS = 106
TCHUNKS = SCHUNKS + 2
IDX_PAD = NW * TCHUNKS * CHUNK
ROWS_PER_SUB = N_PAD // NUM_SUBCORES

BLK = 1280
GRID = N_PAD // BLK


def _sc_mesh():
    return plsc.VectorSubcoreMesh(
        core_axis_name="c", subcore_axis_name="s",
        num_cores=NUM_CORES, num_subcores=NUM_SUBCORES)


def _make_deg_kernel():
    @functools.partial(
        pl.kernel,
        out_type=jax.ShapeDtypeStruct((NUM_CORES, N_PAD, 16), jnp.float32),
        mesh=_sc_mesh(),
        scratch_types=[
            pltpu.VMEM((TCHUNKS, CHUNK), jnp.int32),
            pltpu.VMEM((CHUNK, 16), jnp.float32),
            pltpu.VMEM_SHARED((N_PAD, 16), jnp.float32),
        ],
        compiler_params=pltpu.CompilerParams(use_tc_tiling_on_sc=False),
    )
    def deg_kernel(dst_hbm, ones_hbm, zeros_hbm, out_hbm, dst_v, ones_v, acc):
        c = lax.axis_index("c")
        s = lax.axis_index("s")
        wid = s * NUM_CORES + c
        row0 = s * ROWS_PER_SUB
        pltpu.sync_copy(zeros_hbm.at[pl.ds(row0, ROWS_PER_SUB)],
                        acc.at[pl.ds(row0, ROWS_PER_SUB)])
        pltpu.sync_copy(dst_hbm.at[wid], dst_v)
        pltpu.sync_copy(ones_hbm, ones_v)
        plsc.subcore_barrier()

        def body(j, carry):
            pltpu.sync_copy(ones_v, acc.at[dst_v.at[j]], add=True)
            return carry

        lax.fori_loop(0, SCHUNKS, body, 0)
        plsc.subcore_barrier()
        pltpu.sync_copy(acc.at[pl.ds(row0, ROWS_PER_SUB)],
                        out_hbm.at[c].at[pl.ds(row0, ROWS_PER_SUB)])

    return deg_kernel


def _make_edge_kernel(d):
    @functools.partial(
        pl.kernel,
        out_type=jax.ShapeDtypeStruct((NUM_CORES, N_PAD, d), jnp.float32),
        mesh=_sc_mesh(),
        scratch_types=[
            pltpu.VMEM((TCHUNKS, CHUNK), jnp.int32),
            pltpu.VMEM((TCHUNKS, CHUNK), jnp.int32),
            pltpu.VMEM((CHUNK, d), jnp.float32),
            pltpu.VMEM((CHUNK, d), jnp.float32),
            pltpu.VMEM_SHARED((N_PAD, d), jnp.float32),
            pltpu.SemaphoreType.DMA,
            pltpu.SemaphoreType.DMA,
        ],
        compiler_params=pltpu.CompilerParams(use_tc_tiling_on_sc=False),
    )
    def edge_kernel(table_hbm, src_hbm, dst_hbm, zeros_hbm, out_hbm,
                    src_v, dst_v, rows0, rows1, acc, sem0, sem1):
        c = lax.axis_index("c")
        s = lax.axis_index("s")
        wid = s * NUM_CORES + c
        row0 = s * ROWS_PER_SUB

        @pl.when(c == 0)
        def _():
            pltpu.sync_copy(table_hbm.at[pl.ds(row0, ROWS_PER_SUB)],
                            acc.at[pl.ds(row0, ROWS_PER_SUB)])

        @pl.when(c != 0)
        def _():
            pltpu.sync_copy(zeros_hbm.at[pl.ds(row0, ROWS_PER_SUB)],
                            acc.at[pl.ds(row0, ROWS_PER_SUB)])

        pltpu.sync_copy(src_hbm.at[wid], src_v)
        pltpu.sync_copy(dst_hbm.at[wid], dst_v)
        pltpu.async_copy(table_hbm.at[src_v.at[0]], rows0, sem0)
        pltpu.async_copy(table_hbm.at[src_v.at[1]], rows1, sem1)
        plsc.subcore_barrier()

        def body(i, carry):
            c0 = 2 * i
            c1 = 2 * i + 1
            pltpu.make_async_copy(table_hbm.at[src_v.at[c0]], rows0, sem0).wait()
            pltpu.sync_copy(rows0, acc.at[dst_v.at[c0]], add=True)
            pltpu.async_copy(table_hbm.at[src_v.at[c0 + 2]], rows0, sem0)
            pltpu.make_async_copy(table_hbm.at[src_v.at[c1]], rows1, sem1).wait()
            pltpu.sync_copy(rows1, acc.at[dst_v.at[c1]], add=True)
            pltpu.async_copy(table_hbm.at[src_v.at[c1 + 2]], rows1, sem1)
            return carry

        lax.fori_loop(0, SCHUNKS // 2, body, 0)
        pltpu.make_async_copy(table_hbm.at[src_v.at[SCHUNKS]], rows0, sem0).wait()
        pltpu.make_async_copy(table_hbm.at[src_v.at[SCHUNKS + 1]], rows1, sem1).wait()
        plsc.subcore_barrier()
        pltpu.sync_copy(acc.at[pl.ds(row0, ROWS_PER_SUB)],
                        out_hbm.at[c].at[pl.ds(row0, ROWS_PER_SUB)])

    return edge_kernel


def _dinv_block(deg0, deg1):
    d = deg0[:, 0:1] + deg1[:, 0:1] + 1.0
    return lax.rsqrt(d)


def _mm1_body(x_ref, w_ref, deg0_ref, deg1_ref, o_ref):
    dinv = _dinv_block(deg0_ref[...], deg1_ref[...])
    u = jnp.dot(x_ref[...], w_ref[...], preferred_element_type=jnp.float32)
    o_ref[...] = u * dinv


def _combine_mm2_body(p0_ref, p1_ref, deg0_ref, deg1_ref, b1_ref, w2_ref, o_ref):
    dinv = _dinv_block(deg0_ref[...], deg1_ref[...])
    x1 = jnp.maximum((p0_ref[...] + p1_ref[...]) * dinv + b1_ref[...], 0.0)
    u = jnp.dot(x1, w2_ref[...], preferred_element_type=jnp.float32)
    o_ref[...] = u * dinv


def _pool_body(q0_ref, q1_ref, deg0_ref, deg1_ref, b2_ref, o_ref):
    i = pl.program_id(0)
    dinv = _dinv_block(deg0_ref[...], deg1_ref[...])
    x2 = jnp.maximum((q0_ref[...] + q1_ref[...]) * dinv + b2_ref[...], 0.0)
    rid = i * BLK + lax.broadcasted_iota(jnp.int32, (BLK, EMB), 0)
    x2 = jnp.where(rid < N, x2, 0.0)
    o_ref[...] = jnp.sum(x2, axis=0, keepdims=True).reshape(1, 1, EMB)


def _head_body(s_ref, fw_ref, fb_ref, ow_ref, ob_ref, o_ref):
    pooled = jnp.sum(s_ref[...], axis=0, keepdims=True) * (1.0 / N)
    hid = jnp.dot(pooled, fw_ref[...], preferred_element_type=jnp.float32)
    hid = jnp.maximum(hid + fb_ref[...], 0.0)
    o_ref[...] = jnp.dot(hid, ow_ref[...], preferred_element_type=jnp.float32) + ob_ref[...]


def _row_spec(width):
    return pl.BlockSpec((BLK, width), lambda i: (i, 0))


def _full_spec(shape):
    return pl.BlockSpec(shape, lambda i: (0,) * len(shape))


def kernel(node_features, edge_index, W1, b1, W2, b2, fc_W, fc_b, out_W, out_b):
    x = node_features.reshape(-1, F_IN)
    ei = edge_index.reshape(2, -1).astype(jnp.int32)
    src = ei[0]
    dst = ei[1]

    e_sc = NW * SCHUNKS * CHUNK
    pad_idx = jnp.full((e_sc - E,), N, jnp.int32)
    tail = jnp.full((NW, TCHUNKS - SCHUNKS, CHUNK), N, jnp.int32)
    src_p = jnp.concatenate(
        [jnp.concatenate([src, pad_idx]).reshape(NW, SCHUNKS, CHUNK), tail], axis=1)
    dst_p = jnp.concatenate(
        [jnp.concatenate([dst, pad_idx]).reshape(NW, SCHUNKS, CHUNK), tail], axis=1)

    x_pad = jnp.pad(x, ((0, N_PAD - N), (0, 0)))
    zeros16 = jnp.zeros((N_PAD, 16), jnp.float32)
    zeros128 = jnp.zeros((N_PAD, H1), jnp.float32)
    zeros64 = jnp.zeros((N_PAD, EMB), jnp.float32)
    ones16 = jnp.ones((CHUNK, 16), jnp.float32)

    deg = _make_deg_kernel()(dst_p, ones16, zeros16)
    deg0, deg1 = deg[0], deg[1]

    h1p = pl.pallas_call(
        _mm1_body,
        grid=(GRID,),
        in_specs=[_row_spec(F_IN), _full_spec((F_IN, H1)),
                  _row_spec(16), _row_spec(16)],
        out_specs=_row_spec(H1),
        out_shape=jax.ShapeDtypeStruct((N_PAD, H1), jnp.float32),
    )(x_pad, W1, deg0, deg1)

    p = _make_edge_kernel(H1)(h1p, src_p, dst_p, zeros128)

    h2p = pl.pallas_call(
        _combine_mm2_body,
        grid=(GRID,),
        in_specs=[_row_spec(H1), _row_spec(H1), _row_spec(16), _row_spec(16),
                  _full_spec((1, H1)), _full_spec((H1, EMB))],
        out_specs=_row_spec(EMB),
        out_shape=jax.ShapeDtypeStruct((N_PAD, EMB), jnp.float32),
    )(p[0], p[1], deg0, deg1, b1.reshape(1, H1), W2)

    q = _make_edge_kernel(EMB)(h2p, src_p, dst_p, zeros64)

    part = pl.pallas_call(
        _pool_body,
        grid=(GRID,),
        in_specs=[_row_spec(EMB), _row_spec(EMB), _row_spec(16), _row_spec(16),
                  _full_spec((1, EMB))],
        out_specs=pl.BlockSpec((1, 1, EMB), lambda i: (i, 0, 0)),
        out_shape=jax.ShapeDtypeStruct((GRID, 1, EMB), jnp.float32),
    )(q[0], q[1], deg0, deg1, b2.reshape(1, EMB))
    part = part.reshape(GRID, EMB)

    q_values = pl.pallas_call(
        _head_body,
        in_specs=[pl.BlockSpec((GRID, EMB), lambda: (0, 0)),
                  pl.BlockSpec((EMB, EMB), lambda: (0, 0)),
                  pl.BlockSpec((1, EMB), lambda: (0, 0)),
                  pl.BlockSpec((EMB, ACT), lambda: (0, 0)),
                  pl.BlockSpec((1, ACT), lambda: (0, 0))],
        out_specs=pl.BlockSpec((1, ACT), lambda: (0, 0)),
        out_shape=jax.ShapeDtypeStruct((1, ACT), jnp.float32),
    )(part, fc_W, fc_b.reshape(1, EMB), out_W, out_b.reshape(1, ACT))

    return q_values


# speedup vs baseline: 1.1124x; 1.1124x over previous
import functools

import jax
import jax.numpy as jnp
from jax import lax
from jax.experimental import pallas as pl
from jax.experimental.pallas import tpu as pltpu
from jax.experimental.pallas import tpu_sc as plsc

N = 10000
N_PAD = 10240
E = 320000
F_IN = 128
H1 = 128
EMB = 64
ACT = 18

NUM_CORES = 2
NUM_SUBCORES = 16
NW = NUM_CORES * NUM_SUBCORES
CHUNK = 96
SCHUNKS = 106
TCHUNKS = SCHUNKS + 2
IDX_PAD = NW * TCHUNKS * CHUNK
ROWS_PER_SUB = N_PAD // NUM_SUBCORES

BLK = 1280
GRID = N_PAD // BLK


def _sc_mesh():
    return plsc.VectorSubcoreMesh(
        core_axis_name="c", subcore_axis_name="s",
        num_cores=NUM_CORES, num_subcores=NUM_SUBCORES)


def _make_deg_kernel():
    @functools.partial(
        pl.kernel,
        out_type=jax.ShapeDtypeStruct((NUM_CORES, N_PAD, 16), jnp.float32),
        mesh=_sc_mesh(),
        scratch_types=[
            pltpu.VMEM((TCHUNKS, CHUNK), jnp.int32),
            pltpu.VMEM((CHUNK, 16), jnp.float32),
            pltpu.VMEM_SHARED((N_PAD, 16), jnp.float32),
        ],
        compiler_params=pltpu.CompilerParams(use_tc_tiling_on_sc=False),
    )
    def deg_kernel(dst_hbm, ones_hbm, zeros_hbm, out_hbm, dst_v, ones_v, acc):
        c = lax.axis_index("c")
        s = lax.axis_index("s")
        wid = s * NUM_CORES + c
        row0 = s * ROWS_PER_SUB
        pltpu.sync_copy(zeros_hbm.at[pl.ds(row0, ROWS_PER_SUB)],
                        acc.at[pl.ds(row0, ROWS_PER_SUB)])
        pltpu.sync_copy(dst_hbm.at[wid], dst_v)
        pltpu.sync_copy(ones_hbm, ones_v)
        plsc.subcore_barrier()

        def body(j, carry):
            pltpu.sync_copy(ones_v, acc.at[dst_v.at[j]], add=True)
            return carry

        lax.fori_loop(0, SCHUNKS, body, 0)
        plsc.subcore_barrier()
        pltpu.sync_copy(acc.at[pl.ds(row0, ROWS_PER_SUB)],
                        out_hbm.at[c].at[pl.ds(row0, ROWS_PER_SUB)])

    return deg_kernel


def _make_edge_kernel(d):
    @functools.partial(
        pl.kernel,
        out_type=jax.ShapeDtypeStruct((NUM_CORES, N_PAD, d), jnp.float32),
        mesh=_sc_mesh(),
        scratch_types=[
            pltpu.VMEM((TCHUNKS, CHUNK), jnp.int32),
            pltpu.VMEM((TCHUNKS, CHUNK), jnp.int32),
            pltpu.VMEM((CHUNK, d), jnp.float32),
            pltpu.VMEM((CHUNK, d), jnp.float32),
            pltpu.VMEM_SHARED((N_PAD, d), jnp.float32),
            pltpu.SemaphoreType.DMA,
            pltpu.SemaphoreType.DMA,
        ],
        compiler_params=pltpu.CompilerParams(use_tc_tiling_on_sc=False),
    )
    def edge_kernel(table_hbm, src_hbm, dst_hbm, zeros_hbm, out_hbm,
                    src_v, dst_v, rows0, rows1, acc, sem0, sem1):
        c = lax.axis_index("c")
        s = lax.axis_index("s")
        wid = s * NUM_CORES + c
        row0 = s * ROWS_PER_SUB

        @pl.when(c == 0)
        def _():
            pltpu.sync_copy(table_hbm.at[pl.ds(row0, ROWS_PER_SUB)],
                            acc.at[pl.ds(row0, ROWS_PER_SUB)])

        @pl.when(c != 0)
        def _():
            pltpu.sync_copy(zeros_hbm.at[pl.ds(row0, ROWS_PER_SUB)],
                            acc.at[pl.ds(row0, ROWS_PER_SUB)])

        pltpu.sync_copy(src_hbm.at[wid], src_v)
        pltpu.sync_copy(dst_hbm.at[wid], dst_v)
        pltpu.async_copy(table_hbm.at[src_v.at[0]], rows0, sem0).wait()
        plsc.subcore_barrier()

        def body(i, carry):
            c0 = 2 * i
            c1 = 2 * i + 1
            d1 = pltpu.async_copy(table_hbm.at[src_v.at[c1]], rows1, sem1)
            pltpu.sync_copy(rows0, acc.at[dst_v.at[c0]], add=True)
            d1.wait()
            d0 = pltpu.async_copy(table_hbm.at[src_v.at[c0 + 2]], rows0, sem0)
            pltpu.sync_copy(rows1, acc.at[dst_v.at[c1]], add=True)
            d0.wait()
            return carry

        lax.fori_loop(0, SCHUNKS // 2, body, 0)
        plsc.subcore_barrier()
        pltpu.sync_copy(acc.at[pl.ds(row0, ROWS_PER_SUB)],
                        out_hbm.at[c].at[pl.ds(row0, ROWS_PER_SUB)])

    return edge_kernel


def _dinv_block(deg0, deg1):
    d = deg0[:, 0:1] + deg1[:, 0:1] + 1.0
    return lax.rsqrt(d)


def _mm1_body(x_ref, w_ref, deg0_ref, deg1_ref, o_ref):
    dinv = _dinv_block(deg0_ref[...], deg1_ref[...])
    u = jnp.dot(x_ref[...], w_ref[...], preferred_element_type=jnp.float32)
    o_ref[...] = u * dinv


def _combine_mm2_body(p0_ref, p1_ref, deg0_ref, deg1_ref, b1_ref, w2_ref, o_ref):
    dinv = _dinv_block(deg0_ref[...], deg1_ref[...])
    x1 = jnp.maximum((p0_ref[...] + p1_ref[...]) * dinv + b1_ref[...], 0.0)
    u = jnp.dot(x1, w2_ref[...], preferred_element_type=jnp.float32)
    o_ref[...] = u * dinv


def _pool_body(q0_ref, q1_ref, deg0_ref, deg1_ref, b2_ref, o_ref):
    i = pl.program_id(0)
    dinv = _dinv_block(deg0_ref[...], deg1_ref[...])
    x2 = jnp.maximum((q0_ref[...] + q1_ref[...]) * dinv + b2_ref[...], 0.0)
    rid = i * BLK + lax.broadcasted_iota(jnp.int32, (BLK, EMB), 0)
    x2 = jnp.where(rid < N, x2, 0.0)
    o_ref[...] = jnp.sum(x2, axis=0, keepdims=True).reshape(1, 1, EMB)


def _head_body(s_ref, fw_ref, fb_ref, ow_ref, ob_ref, o_ref):
    pooled = jnp.sum(s_ref[...], axis=0, keepdims=True) * (1.0 / N)
    hid = jnp.dot(pooled, fw_ref[...], preferred_element_type=jnp.float32)
    hid = jnp.maximum(hid + fb_ref[...], 0.0)
    o_ref[...] = jnp.dot(hid, ow_ref[...], preferred_element_type=jnp.float32) + ob_ref[...]


def _row_spec(width):
    return pl.BlockSpec((BLK, width), lambda i: (i, 0))


def _full_spec(shape):
    return pl.BlockSpec(shape, lambda i: (0,) * len(shape))


def kernel(node_features, edge_index, W1, b1, W2, b2, fc_W, fc_b, out_W, out_b):
    x = node_features.reshape(-1, F_IN)
    ei = edge_index.reshape(2, -1).astype(jnp.int32)
    src = ei[0]
    dst = ei[1]

    e_sc = NW * SCHUNKS * CHUNK
    pad_idx = jnp.full((e_sc - E,), N, jnp.int32)
    tail = jnp.full((NW, TCHUNKS - SCHUNKS, CHUNK), N, jnp.int32)
    src_p = jnp.concatenate(
        [jnp.concatenate([src, pad_idx]).reshape(NW, SCHUNKS, CHUNK), tail], axis=1)
    dst_p = jnp.concatenate(
        [jnp.concatenate([dst, pad_idx]).reshape(NW, SCHUNKS, CHUNK), tail], axis=1)

    x_pad = jnp.pad(x, ((0, N_PAD - N), (0, 0)))
    zeros16 = jnp.zeros((N_PAD, 16), jnp.float32)
    zeros128 = jnp.zeros((N_PAD, H1), jnp.float32)
    zeros64 = jnp.zeros((N_PAD, EMB), jnp.float32)
    ones16 = jnp.ones((CHUNK, 16), jnp.float32)

    deg = _make_deg_kernel()(dst_p, ones16, zeros16)
    deg0, deg1 = deg[0], deg[1]

    h1p = pl.pallas_call(
        _mm1_body,
        grid=(GRID,),
        in_specs=[_row_spec(F_IN), _full_spec((F_IN, H1)),
                  _row_spec(16), _row_spec(16)],
        out_specs=_row_spec(H1),
        out_shape=jax.ShapeDtypeStruct((N_PAD, H1), jnp.float32),
    )(x_pad, W1, deg0, deg1)

    p = _make_edge_kernel(H1)(h1p, src_p, dst_p, zeros128)

    h2p = pl.pallas_call(
        _combine_mm2_body,
        grid=(GRID,),
        in_specs=[_row_spec(H1), _row_spec(H1), _row_spec(16), _row_spec(16),
                  _full_spec((1, H1)), _full_spec((H1, EMB))],
        out_specs=_row_spec(EMB),
        out_shape=jax.ShapeDtypeStruct((N_PAD, EMB), jnp.float32),
    )(p[0], p[1], deg0, deg1, b1.reshape(1, H1), W2)

    q = _make_edge_kernel(EMB)(h2p, src_p, dst_p, zeros64)

    part = pl.pallas_call(
        _pool_body,
        grid=(GRID,),
        in_specs=[_row_spec(EMB), _row_spec(EMB), _row_spec(16), _row_spec(16),
                  _full_spec((1, EMB))],
        out_specs=pl.BlockSpec((1, 1, EMB), lambda i: (i, 0, 0)),
        out_shape=jax.ShapeDtypeStruct((GRID, 1, EMB), jnp.float32),
    )(q[0], q[1], deg0, deg1, b2.reshape(1, EMB))
    part = part.reshape(GRID, EMB)

    q_values = pl.pallas_call(
        _head_body,
        in_specs=[pl.BlockSpec((GRID, EMB), lambda: (0, 0)),
                  pl.BlockSpec((EMB, EMB), lambda: (0, 0)),
                  pl.BlockSpec((1, EMB), lambda: (0, 0)),
                  pl.BlockSpec((EMB, ACT), lambda: (0, 0)),
                  pl.BlockSpec((1, ACT), lambda: (0, 0))],
        out_specs=pl.BlockSpec((1, ACT), lambda: (0, 0)),
        out_shape=jax.ShapeDtypeStruct((1, ACT), jnp.float32),
    )(part, fc_W, fc_b.reshape(1, EMB), out_W, out_b.reshape(1, ACT))

    return q_values


# speedup vs baseline: 1.5813x; 1.4215x over previous
import functools

import jax
import jax.numpy as jnp
from jax import lax
from jax.experimental import pallas as pl
from jax.experimental.pallas import tpu as pltpu
from jax.experimental.pallas import tpu_sc as plsc

N = 10000
N_PAD = 10240
E = 320000
F_IN = 128
H1 = 128
EMB = 64
ACT = 18

NUM_CORES = 2
NUM_SUBCORES = 16
NW = NUM_CORES * NUM_SUBCORES
CHUNK = 128
SCHUNKS = 79
TCHUNKS = SCHUNKS
IDX_PAD = NW * TCHUNKS * CHUNK
ROWS_PER_SUB = N_PAD // NUM_SUBCORES

BLK = 1280
GRID = N_PAD // BLK


def _sc_mesh():
    return plsc.VectorSubcoreMesh(
        core_axis_name="c", subcore_axis_name="s",
        num_cores=NUM_CORES, num_subcores=NUM_SUBCORES)


def _make_deg_kernel():
    @functools.partial(
        pl.kernel,
        out_type=jax.ShapeDtypeStruct((NUM_CORES, N_PAD, 16), jnp.float32),
        mesh=_sc_mesh(),
        scratch_types=[
            pltpu.VMEM((TCHUNKS, CHUNK), jnp.int32),
            pltpu.VMEM((CHUNK, 16), jnp.float32),
            pltpu.VMEM_SHARED((N_PAD, 16), jnp.float32),
        ],
        compiler_params=pltpu.CompilerParams(use_tc_tiling_on_sc=False),
    )
    def deg_kernel(dst_hbm, ones_hbm, zeros_hbm, out_hbm, dst_v, ones_v, acc):
        c = lax.axis_index("c")
        s = lax.axis_index("s")
        wid = s * NUM_CORES + c
        row0 = s * ROWS_PER_SUB
        pltpu.sync_copy(zeros_hbm.at[pl.ds(row0, ROWS_PER_SUB)],
                        acc.at[pl.ds(row0, ROWS_PER_SUB)])
        pltpu.sync_copy(dst_hbm.at[wid], dst_v)
        pltpu.sync_copy(ones_hbm, ones_v)
        plsc.subcore_barrier()

        def body(j, carry):
            pltpu.sync_copy(ones_v, acc.at[dst_v.at[j]], add=True)
            return carry

        lax.fori_loop(0, SCHUNKS, body, 0)
        plsc.subcore_barrier()
        pltpu.sync_copy(acc.at[pl.ds(row0, ROWS_PER_SUB)],
                        out_hbm.at[c].at[pl.ds(row0, ROWS_PER_SUB)])

    return deg_kernel


def _make_edge_kernel(d):
    @functools.partial(
        pl.kernel,
        out_type=jax.ShapeDtypeStruct((NUM_CORES, N_PAD, d), jnp.float32),
        mesh=_sc_mesh(),
        scratch_types=[
            pltpu.VMEM((TCHUNKS, CHUNK), jnp.int32),
            pltpu.VMEM((TCHUNKS, CHUNK), jnp.int32),
            pltpu.VMEM((CHUNK, d), jnp.float32),
            pltpu.VMEM_SHARED((N_PAD, d), jnp.float32),
            pltpu.SemaphoreType.DMA,
        ],
        compiler_params=pltpu.CompilerParams(use_tc_tiling_on_sc=False),
    )
    def edge_kernel(table_hbm, src_hbm, dst_hbm, zeros_hbm, out_hbm,
                    src_v, dst_v, rows_v, acc, sem):
        c = lax.axis_index("c")
        s = lax.axis_index("s")
        wid = s * NUM_CORES + c
        row0 = s * ROWS_PER_SUB

        @pl.when(c == 0)
        def _():
            pltpu.sync_copy(table_hbm.at[pl.ds(row0, ROWS_PER_SUB)],
                            acc.at[pl.ds(row0, ROWS_PER_SUB)])

        @pl.when(c != 0)
        def _():
            pltpu.sync_copy(zeros_hbm.at[pl.ds(row0, ROWS_PER_SUB)],
                            acc.at[pl.ds(row0, ROWS_PER_SUB)])

        pltpu.sync_copy(src_hbm.at[wid], src_v)
        pltpu.sync_copy(dst_hbm.at[wid], dst_v)
        plsc.subcore_barrier()

        def body(j, carry):
            pltpu.async_copy(table_hbm.at[src_v.at[j]], rows_v, sem).wait()
            pltpu.sync_copy(rows_v, acc.at[dst_v.at[j]], add=True)
            return carry

        lax.fori_loop(0, SCHUNKS, body, 0)
        plsc.subcore_barrier()
        pltpu.sync_copy(acc.at[pl.ds(row0, ROWS_PER_SUB)],
                        out_hbm.at[c].at[pl.ds(row0, ROWS_PER_SUB)])

    return edge_kernel


def _dinv_block(deg0, deg1):
    d = deg0[:, 0:1] + deg1[:, 0:1] + 1.0
    return lax.rsqrt(d)


def _mm1_body(x_ref, w_ref, deg0_ref, deg1_ref, o_ref):
    dinv = _dinv_block(deg0_ref[...], deg1_ref[...])
    u = jnp.dot(x_ref[...], w_ref[...], preferred_element_type=jnp.float32)
    o_ref[...] = u * dinv


def _combine_mm2_body(p0_ref, p1_ref, deg0_ref, deg1_ref, b1_ref, w2_ref, o_ref):
    dinv = _dinv_block(deg0_ref[...], deg1_ref[...])
    x1 = jnp.maximum((p0_ref[...] + p1_ref[...]) * dinv + b1_ref[...], 0.0)
    u = jnp.dot(x1, w2_ref[...], preferred_element_type=jnp.float32)
    o_ref[...] = u * dinv


def _pool_body(q0_ref, q1_ref, deg0_ref, deg1_ref, b2_ref, o_ref):
    i = pl.program_id(0)
    dinv = _dinv_block(deg0_ref[...], deg1_ref[...])
    x2 = jnp.maximum((q0_ref[...] + q1_ref[...]) * dinv + b2_ref[...], 0.0)
    rid = i * BLK + lax.broadcasted_iota(jnp.int32, (BLK, EMB), 0)
    x2 = jnp.where(rid < N, x2, 0.0)
    o_ref[...] = jnp.sum(x2, axis=0, keepdims=True).reshape(1, 1, EMB)


def _head_body(s_ref, fw_ref, fb_ref, ow_ref, ob_ref, o_ref):
    pooled = jnp.sum(s_ref[...], axis=0, keepdims=True) * (1.0 / N)
    hid = jnp.dot(pooled, fw_ref[...], preferred_element_type=jnp.float32)
    hid = jnp.maximum(hid + fb_ref[...], 0.0)
    o_ref[...] = jnp.dot(hid, ow_ref[...], preferred_element_type=jnp.float32) + ob_ref[...]


def _row_spec(width):
    return pl.BlockSpec((BLK, width), lambda i: (i, 0))


def _full_spec(shape):
    return pl.BlockSpec(shape, lambda i: (0,) * len(shape))


def kernel(node_features, edge_index, W1, b1, W2, b2, fc_W, fc_b, out_W, out_b):
    x = node_features.reshape(-1, F_IN)
    ei = edge_index.reshape(2, -1).astype(jnp.int32)
    src = ei[0]
    dst = ei[1]

    e_sc = NW * SCHUNKS * CHUNK
    pad_idx = jnp.full((e_sc - E,), N, jnp.int32)
    src_p = jnp.concatenate([src, pad_idx]).reshape(NW, SCHUNKS, CHUNK)
    dst_p = jnp.concatenate([dst, pad_idx]).reshape(NW, SCHUNKS, CHUNK)

    x_pad = jnp.pad(x, ((0, N_PAD - N), (0, 0)))
    zeros16 = jnp.zeros((N_PAD, 16), jnp.float32)
    zeros128 = jnp.zeros((N_PAD, H1), jnp.float32)
    zeros64 = jnp.zeros((N_PAD, EMB), jnp.float32)
    ones16 = jnp.ones((CHUNK, 16), jnp.float32)

    deg = _make_deg_kernel()(dst_p, ones16, zeros16)
    deg0, deg1 = deg[0], deg[1]

    h1p = pl.pallas_call(
        _mm1_body,
        grid=(GRID,),
        in_specs=[_row_spec(F_IN), _full_spec((F_IN, H1)),
                  _row_spec(16), _row_spec(16)],
        out_specs=_row_spec(H1),
        out_shape=jax.ShapeDtypeStruct((N_PAD, H1), jnp.float32),
    )(x_pad, W1, deg0, deg1)

    p = _make_edge_kernel(H1)(h1p, src_p, dst_p, zeros128)

    h2p = pl.pallas_call(
        _combine_mm2_body,
        grid=(GRID,),
        in_specs=[_row_spec(H1), _row_spec(H1), _row_spec(16), _row_spec(16),
                  _full_spec((1, H1)), _full_spec((H1, EMB))],
        out_specs=_row_spec(EMB),
        out_shape=jax.ShapeDtypeStruct((N_PAD, EMB), jnp.float32),
    )(p[0], p[1], deg0, deg1, b1.reshape(1, H1), W2)

    q = _make_edge_kernel(EMB)(h2p, src_p, dst_p, zeros64)

    part = pl.pallas_call(
        _pool_body,
        grid=(GRID,),
        in_specs=[_row_spec(EMB), _row_spec(EMB), _row_spec(16), _row_spec(16),
                  _full_spec((1, EMB))],
        out_specs=pl.BlockSpec((1, 1, EMB), lambda i: (i, 0, 0)),
        out_shape=jax.ShapeDtypeStruct((GRID, 1, EMB), jnp.float32),
    )(q[0], q[1], deg0, deg1, b2.reshape(1, EMB))
    part = part.reshape(GRID, EMB)

    q_values = pl.pallas_call(
        _head_body,
        in_specs=[pl.BlockSpec((GRID, EMB), lambda: (0, 0)),
                  pl.BlockSpec((EMB, EMB), lambda: (0, 0)),
                  pl.BlockSpec((1, EMB), lambda: (0, 0)),
                  pl.BlockSpec((EMB, ACT), lambda: (0, 0)),
                  pl.BlockSpec((1, ACT), lambda: (0, 0))],
        out_specs=pl.BlockSpec((1, ACT), lambda: (0, 0)),
        out_shape=jax.ShapeDtypeStruct((1, ACT), jnp.float32),
    )(part, fc_W, fc_b.reshape(1, EMB), out_W, out_b.reshape(1, ACT))

    return q_values


# speedup vs baseline: 2.4343x; 1.5394x over previous
import functools

import jax
import jax.numpy as jnp
from jax import lax
from jax.experimental import pallas as pl
from jax.experimental.pallas import tpu as pltpu
from jax.experimental.pallas import tpu_sc as plsc

N = 10000
N_PAD = 10240
E = 320000
F_IN = 128
H1 = 128
EMB = 64
ACT = 18

NUM_CORES = 2
NUM_SUBCORES = 16
NW = NUM_CORES * NUM_SUBCORES
CHUNK = 128
SCHUNKS = 79
TCHUNKS = SCHUNKS
IDX_PAD = NW * TCHUNKS * CHUNK
ROWS_PER_SUB = N_PAD // NUM_SUBCORES

BLK = 1280
GRID = N_PAD // BLK


def _sc_mesh():
    return plsc.VectorSubcoreMesh(
        core_axis_name="c", subcore_axis_name="s",
        num_cores=NUM_CORES, num_subcores=NUM_SUBCORES)


def _make_deg_kernel():
    @functools.partial(
        pl.kernel,
        out_type=jax.ShapeDtypeStruct((NUM_CORES, N_PAD, 16), jnp.float32),
        mesh=_sc_mesh(),
        scratch_types=[
            pltpu.VMEM((TCHUNKS, CHUNK), jnp.int32),
            pltpu.VMEM((CHUNK, 16), jnp.float32),
            pltpu.VMEM_SHARED((N_PAD, 16), jnp.float32),
        ],
        compiler_params=pltpu.CompilerParams(use_tc_tiling_on_sc=False),
    )
    def deg_kernel(dst_hbm, ones_hbm, zeros_hbm, out_hbm, dst_v, ones_v, acc):
        c = lax.axis_index("c")
        s = lax.axis_index("s")
        wid = s * NUM_CORES + c
        row0 = s * ROWS_PER_SUB
        pltpu.sync_copy(zeros_hbm.at[pl.ds(row0, ROWS_PER_SUB)],
                        acc.at[pl.ds(row0, ROWS_PER_SUB)])
        pltpu.sync_copy(dst_hbm.at[wid], dst_v)
        pltpu.sync_copy(ones_hbm, ones_v)
        plsc.subcore_barrier()

        def body(j, carry):
            pltpu.sync_copy(ones_v, acc.at[dst_v.at[j]], add=True)
            return carry

        lax.fori_loop(0, SCHUNKS, body, 0)
        plsc.subcore_barrier()
        pltpu.sync_copy(acc.at[pl.ds(row0, ROWS_PER_SUB)],
                        out_hbm.at[c].at[pl.ds(row0, ROWS_PER_SUB)])

    return deg_kernel


def _make_edge_kernel(d, dtype=jnp.bfloat16):
    @functools.partial(
        pl.kernel,
        out_type=jax.ShapeDtypeStruct((NUM_CORES, N_PAD, d), dtype),
        mesh=_sc_mesh(),
        scratch_types=[
            pltpu.VMEM((TCHUNKS, CHUNK), jnp.int32),
            pltpu.VMEM((TCHUNKS, CHUNK), jnp.int32),
            pltpu.VMEM((CHUNK, d), dtype),
            pltpu.VMEM_SHARED((N_PAD, d), dtype),
            pltpu.SemaphoreType.DMA,
        ],
        compiler_params=pltpu.CompilerParams(use_tc_tiling_on_sc=False),
    )
    def edge_kernel(table_hbm, src_hbm, dst_hbm, zeros_hbm, out_hbm,
                    src_v, dst_v, rows_v, acc, sem):
        c = lax.axis_index("c")
        s = lax.axis_index("s")
        wid = s * NUM_CORES + c
        row0 = s * ROWS_PER_SUB

        @pl.when(c == 0)
        def _():
            pltpu.sync_copy(table_hbm.at[pl.ds(row0, ROWS_PER_SUB)],
                            acc.at[pl.ds(row0, ROWS_PER_SUB)])

        @pl.when(c != 0)
        def _():
            pltpu.sync_copy(zeros_hbm.at[pl.ds(row0, ROWS_PER_SUB)],
                            acc.at[pl.ds(row0, ROWS_PER_SUB)])

        pltpu.sync_copy(src_hbm.at[wid], src_v)
        pltpu.sync_copy(dst_hbm.at[wid], dst_v)
        plsc.subcore_barrier()

        def body(j, carry):
            pltpu.async_copy(table_hbm.at[src_v.at[j]], rows_v, sem).wait()
            pltpu.sync_copy(rows_v, acc.at[dst_v.at[j]], add=True)
            return carry

        lax.fori_loop(0, SCHUNKS, body, 0)
        plsc.subcore_barrier()
        pltpu.sync_copy(acc.at[pl.ds(row0, ROWS_PER_SUB)],
                        out_hbm.at[c].at[pl.ds(row0, ROWS_PER_SUB)])

    return edge_kernel


def _dinv_block(deg0, deg1):
    d = deg0[:, 0:1] + deg1[:, 0:1] + 1.0
    return lax.rsqrt(d)


def _mm1_body(x_ref, w_ref, deg0_ref, deg1_ref, o_ref):
    dinv = _dinv_block(deg0_ref[...], deg1_ref[...])
    u = jnp.dot(x_ref[...], w_ref[...], preferred_element_type=jnp.float32)
    o_ref[...] = (u * dinv).astype(o_ref.dtype)


def _combine_mm2_body(p0_ref, p1_ref, deg0_ref, deg1_ref, b1_ref, w2_ref, o_ref):
    dinv = _dinv_block(deg0_ref[...], deg1_ref[...])
    psum = p0_ref[...].astype(jnp.float32) + p1_ref[...].astype(jnp.float32)
    x1 = jnp.maximum(psum * dinv + b1_ref[...], 0.0)
    u = jnp.dot(x1, w2_ref[...], preferred_element_type=jnp.float32)
    o_ref[...] = (u * dinv).astype(o_ref.dtype)


def _pool_body(q0_ref, q1_ref, deg0_ref, deg1_ref, b2_ref, o_ref):
    i = pl.program_id(0)
    dinv = _dinv_block(deg0_ref[...], deg1_ref[...])
    qsum = q0_ref[...].astype(jnp.float32) + q1_ref[...].astype(jnp.float32)
    x2 = jnp.maximum(qsum * dinv + b2_ref[...], 0.0)
    rid = i * BLK + lax.broadcasted_iota(jnp.int32, (BLK, EMB), 0)
    x2 = jnp.where(rid < N, x2, 0.0)
    o_ref[...] = jnp.sum(x2, axis=0, keepdims=True).reshape(1, 1, EMB)


def _head_body(s_ref, fw_ref, fb_ref, ow_ref, ob_ref, o_ref):
    pooled = jnp.sum(s_ref[...], axis=0, keepdims=True) * (1.0 / N)
    hid = jnp.dot(pooled, fw_ref[...], preferred_element_type=jnp.float32)
    hid = jnp.maximum(hid + fb_ref[...], 0.0)
    o_ref[...] = jnp.dot(hid, ow_ref[...], preferred_element_type=jnp.float32) + ob_ref[...]


def _row_spec(width):
    return pl.BlockSpec((BLK, width), lambda i: (i, 0))


def _full_spec(shape):
    return pl.BlockSpec(shape, lambda i: (0,) * len(shape))


def kernel(node_features, edge_index, W1, b1, W2, b2, fc_W, fc_b, out_W, out_b):
    x = node_features.reshape(-1, F_IN)
    ei = edge_index.reshape(2, -1).astype(jnp.int32)
    src = ei[0]
    dst = ei[1]

    e_sc = NW * SCHUNKS * CHUNK
    pad_idx = jnp.full((e_sc - E,), N, jnp.int32)
    src_p = jnp.concatenate([src, pad_idx]).reshape(NW, SCHUNKS, CHUNK)
    dst_p = jnp.concatenate([dst, pad_idx]).reshape(NW, SCHUNKS, CHUNK)

    x_pad = jnp.pad(x, ((0, N_PAD - N), (0, 0)))
    zeros16 = jnp.zeros((N_PAD, 16), jnp.float32)
    zeros128 = jnp.zeros((N_PAD, H1), jnp.bfloat16)
    zeros64 = jnp.zeros((N_PAD, EMB), jnp.bfloat16)
    ones16 = jnp.ones((CHUNK, 16), jnp.float32)

    deg = _make_deg_kernel()(dst_p, ones16, zeros16)
    deg0, deg1 = deg[0], deg[1]

    h1p = pl.pallas_call(
        _mm1_body,
        grid=(GRID,),
        in_specs=[_row_spec(F_IN), _full_spec((F_IN, H1)),
                  _row_spec(16), _row_spec(16)],
        out_specs=_row_spec(H1),
        out_shape=jax.ShapeDtypeStruct((N_PAD, H1), jnp.bfloat16),
    )(x_pad, W1, deg0, deg1)

    p = _make_edge_kernel(H1)(h1p, src_p, dst_p, zeros128)

    h2p = pl.pallas_call(
        _combine_mm2_body,
        grid=(GRID,),
        in_specs=[_row_spec(H1), _row_spec(H1), _row_spec(16), _row_spec(16),
                  _full_spec((1, H1)), _full_spec((H1, EMB))],
        out_specs=_row_spec(EMB),
        out_shape=jax.ShapeDtypeStruct((N_PAD, EMB), jnp.bfloat16),
    )(p[0], p[1], deg0, deg1, b1.reshape(1, H1), W2)

    q = _make_edge_kernel(EMB)(h2p, src_p, dst_p, zeros64)

    part = pl.pallas_call(
        _pool_body,
        grid=(GRID,),
        in_specs=[_row_spec(EMB), _row_spec(EMB), _row_spec(16), _row_spec(16),
                  _full_spec((1, EMB))],
        out_specs=pl.BlockSpec((1, 1, EMB), lambda i: (i, 0, 0)),
        out_shape=jax.ShapeDtypeStruct((GRID, 1, EMB), jnp.float32),
    )(q[0], q[1], deg0, deg1, b2.reshape(1, EMB))
    part = part.reshape(GRID, EMB)

    q_values = pl.pallas_call(
        _head_body,
        in_specs=[pl.BlockSpec((GRID, EMB), lambda: (0, 0)),
                  pl.BlockSpec((EMB, EMB), lambda: (0, 0)),
                  pl.BlockSpec((1, EMB), lambda: (0, 0)),
                  pl.BlockSpec((EMB, ACT), lambda: (0, 0)),
                  pl.BlockSpec((1, ACT), lambda: (0, 0))],
        out_specs=pl.BlockSpec((1, ACT), lambda: (0, 0)),
        out_shape=jax.ShapeDtypeStruct((1, ACT), jnp.float32),
    )(part, fc_W, fc_b.reshape(1, EMB), out_W, out_b.reshape(1, ACT))

    return q_values


# speedup vs baseline: 3.4340x; 1.4107x over previous
import functools

import jax
import jax.numpy as jnp
from jax import lax
from jax.experimental import pallas as pl
from jax.experimental.pallas import tpu as pltpu
from jax.experimental.pallas import tpu_sc as plsc

N = 10000
N_PAD = 10240
E = 320000
F_IN = 128
H1 = 128
EMB = 64
ACT = 18

NUM_CORES = 2
NUM_SUBCORES = 16
NW = NUM_CORES * NUM_SUBCORES
CHUNK = 128
SCHUNKS = 79
TCHUNKS = SCHUNKS
IDX_PAD = NW * TCHUNKS * CHUNK
ROWS_PER_SUB = N_PAD // NUM_SUBCORES

BLK = 1280
GRID = N_PAD // BLK


def _sc_mesh():
    return plsc.VectorSubcoreMesh(
        core_axis_name="c", subcore_axis_name="s",
        num_cores=NUM_CORES, num_subcores=NUM_SUBCORES)


def _make_deg_kernel():
    @functools.partial(
        pl.kernel,
        out_type=jax.ShapeDtypeStruct((NUM_CORES, N_PAD, 16), jnp.float32),
        mesh=_sc_mesh(),
        scratch_types=[
            pltpu.VMEM((TCHUNKS, CHUNK), jnp.int32),
            pltpu.VMEM((CHUNK, 16), jnp.float32),
            pltpu.VMEM_SHARED((N_PAD, 16), jnp.float32),
        ],
        compiler_params=pltpu.CompilerParams(use_tc_tiling_on_sc=False),
    )
    def deg_kernel(dst_hbm, ones_hbm, zeros_hbm, out_hbm, dst_v, ones_v, acc):
        c = lax.axis_index("c")
        s = lax.axis_index("s")
        wid = s * NUM_CORES + c
        row0 = s * ROWS_PER_SUB
        pltpu.sync_copy(zeros_hbm.at[pl.ds(row0, ROWS_PER_SUB)],
                        acc.at[pl.ds(row0, ROWS_PER_SUB)])
        pltpu.sync_copy(dst_hbm.at[wid], dst_v)
        pltpu.sync_copy(ones_hbm, ones_v)
        plsc.subcore_barrier()

        def body(j, carry):
            pltpu.sync_copy(ones_v, acc.at[dst_v.at[j]], add=True)
            return carry

        lax.fori_loop(0, SCHUNKS, body, 0)
        plsc.subcore_barrier()
        pltpu.sync_copy(acc.at[pl.ds(row0, ROWS_PER_SUB)],
                        out_hbm.at[c].at[pl.ds(row0, ROWS_PER_SUB)])

    return deg_kernel


def _make_edge_kernel(d, dtype=jnp.bfloat16):
    @functools.partial(
        pl.kernel,
        out_type=jax.ShapeDtypeStruct((NUM_CORES, N_PAD, d), dtype),
        mesh=_sc_mesh(),
        scratch_types=[
            pltpu.VMEM((TCHUNKS, CHUNK), jnp.int32),
            pltpu.VMEM((TCHUNKS, CHUNK), jnp.int32),
            pltpu.VMEM((CHUNK, d), dtype),
            pltpu.VMEM_SHARED((N_PAD, d), dtype),
            pltpu.VMEM_SHARED((N_PAD, d), dtype),
            pltpu.SemaphoreType.DMA,
        ],
        compiler_params=pltpu.CompilerParams(use_tc_tiling_on_sc=False),
    )
    def edge_kernel(table_hbm, src_hbm, dst_hbm, zeros_hbm, out_hbm,
                    src_v, dst_v, rows_v, acc, table_s, sem):
        c = lax.axis_index("c")
        s = lax.axis_index("s")
        wid = s * NUM_CORES + c
        row0 = s * ROWS_PER_SUB

        pltpu.sync_copy(table_hbm.at[pl.ds(row0, ROWS_PER_SUB)],
                        table_s.at[pl.ds(row0, ROWS_PER_SUB)])

        @pl.when(c == 0)
        def _():
            pltpu.sync_copy(table_hbm.at[pl.ds(row0, ROWS_PER_SUB)],
                            acc.at[pl.ds(row0, ROWS_PER_SUB)])

        @pl.when(c != 0)
        def _():
            pltpu.sync_copy(zeros_hbm.at[pl.ds(row0, ROWS_PER_SUB)],
                            acc.at[pl.ds(row0, ROWS_PER_SUB)])

        pltpu.sync_copy(src_hbm.at[wid], src_v)
        pltpu.sync_copy(dst_hbm.at[wid], dst_v)
        plsc.subcore_barrier()

        def body(j, carry):
            pltpu.sync_copy(table_s.at[src_v.at[j]], rows_v)
            pltpu.sync_copy(rows_v, acc.at[dst_v.at[j]], add=True)
            return carry

        lax.fori_loop(0, SCHUNKS, body, 0)
        plsc.subcore_barrier()
        pltpu.sync_copy(acc.at[pl.ds(row0, ROWS_PER_SUB)],
                        out_hbm.at[c].at[pl.ds(row0, ROWS_PER_SUB)])

    return edge_kernel


def _dinv_block(deg0, deg1):
    d = deg0[:, 0:1] + deg1[:, 0:1] + 1.0
    return lax.rsqrt(d)


def _mm1_body(x_ref, w_ref, deg0_ref, deg1_ref, o_ref):
    dinv = _dinv_block(deg0_ref[...], deg1_ref[...])
    u = jnp.dot(x_ref[...], w_ref[...], preferred_element_type=jnp.float32)
    o_ref[...] = (u * dinv).astype(o_ref.dtype)


def _combine_mm2_body(p0_ref, p1_ref, deg0_ref, deg1_ref, b1_ref, w2_ref, o_ref):
    dinv = _dinv_block(deg0_ref[...], deg1_ref[...])
    psum = p0_ref[...].astype(jnp.float32) + p1_ref[...].astype(jnp.float32)
    x1 = jnp.maximum(psum * dinv + b1_ref[...], 0.0)
    u = jnp.dot(x1, w2_ref[...], preferred_element_type=jnp.float32)
    o_ref[...] = (u * dinv).astype(o_ref.dtype)


def _pool_body(q0_ref, q1_ref, deg0_ref, deg1_ref, b2_ref, o_ref):
    i = pl.program_id(0)
    dinv = _dinv_block(deg0_ref[...], deg1_ref[...])
    qsum = q0_ref[...].astype(jnp.float32) + q1_ref[...].astype(jnp.float32)
    x2 = jnp.maximum(qsum * dinv + b2_ref[...], 0.0)
    rid = i * BLK + lax.broadcasted_iota(jnp.int32, (BLK, EMB), 0)
    x2 = jnp.where(rid < N, x2, 0.0)
    o_ref[...] = jnp.sum(x2, axis=0, keepdims=True).reshape(1, 1, EMB)


def _head_body(s_ref, fw_ref, fb_ref, ow_ref, ob_ref, o_ref):
    pooled = jnp.sum(s_ref[...], axis=0, keepdims=True) * (1.0 / N)
    hid = jnp.dot(pooled, fw_ref[...], preferred_element_type=jnp.float32)
    hid = jnp.maximum(hid + fb_ref[...], 0.0)
    o_ref[...] = jnp.dot(hid, ow_ref[...], preferred_element_type=jnp.float32) + ob_ref[...]


def _row_spec(width):
    return pl.BlockSpec((BLK, width), lambda i: (i, 0))


def _full_spec(shape):
    return pl.BlockSpec(shape, lambda i: (0,) * len(shape))


def kernel(node_features, edge_index, W1, b1, W2, b2, fc_W, fc_b, out_W, out_b):
    x = node_features.reshape(-1, F_IN)
    ei = edge_index.reshape(2, -1).astype(jnp.int32)
    src = ei[0]
    dst = ei[1]

    e_sc = NW * SCHUNKS * CHUNK
    pad_idx = jnp.full((e_sc - E,), N, jnp.int32)
    src_p = jnp.concatenate([src, pad_idx]).reshape(NW, SCHUNKS, CHUNK)
    dst_p = jnp.concatenate([dst, pad_idx]).reshape(NW, SCHUNKS, CHUNK)

    x_pad = jnp.pad(x, ((0, N_PAD - N), (0, 0)))
    zeros16 = jnp.zeros((N_PAD, 16), jnp.float32)
    zeros128 = jnp.zeros((N_PAD, H1), jnp.bfloat16)
    zeros64 = jnp.zeros((N_PAD, EMB), jnp.bfloat16)
    ones16 = jnp.ones((CHUNK, 16), jnp.float32)

    deg = _make_deg_kernel()(dst_p, ones16, zeros16)
    deg0, deg1 = deg[0], deg[1]

    h1p = pl.pallas_call(
        _mm1_body,
        grid=(GRID,),
        in_specs=[_row_spec(F_IN), _full_spec((F_IN, H1)),
                  _row_spec(16), _row_spec(16)],
        out_specs=_row_spec(H1),
        out_shape=jax.ShapeDtypeStruct((N_PAD, H1), jnp.bfloat16),
    )(x_pad, W1, deg0, deg1)

    p = _make_edge_kernel(H1)(h1p, src_p, dst_p, zeros128)

    h2p = pl.pallas_call(
        _combine_mm2_body,
        grid=(GRID,),
        in_specs=[_row_spec(H1), _row_spec(H1), _row_spec(16), _row_spec(16),
                  _full_spec((1, H1)), _full_spec((H1, EMB))],
        out_specs=_row_spec(EMB),
        out_shape=jax.ShapeDtypeStruct((N_PAD, EMB), jnp.bfloat16),
    )(p[0], p[1], deg0, deg1, b1.reshape(1, H1), W2)

    q = _make_edge_kernel(EMB)(h2p, src_p, dst_p, zeros64)

    part = pl.pallas_call(
        _pool_body,
        grid=(GRID,),
        in_specs=[_row_spec(EMB), _row_spec(EMB), _row_spec(16), _row_spec(16),
                  _full_spec((1, EMB))],
        out_specs=pl.BlockSpec((1, 1, EMB), lambda i: (i, 0, 0)),
        out_shape=jax.ShapeDtypeStruct((GRID, 1, EMB), jnp.float32),
    )(q[0], q[1], deg0, deg1, b2.reshape(1, EMB))
    part = part.reshape(GRID, EMB)

    q_values = pl.pallas_call(
        _head_body,
        in_specs=[pl.BlockSpec((GRID, EMB), lambda: (0, 0)),
                  pl.BlockSpec((EMB, EMB), lambda: (0, 0)),
                  pl.BlockSpec((1, EMB), lambda: (0, 0)),
                  pl.BlockSpec((EMB, ACT), lambda: (0, 0)),
                  pl.BlockSpec((1, ACT), lambda: (0, 0))],
        out_specs=pl.BlockSpec((1, ACT), lambda: (0, 0)),
        out_shape=jax.ShapeDtypeStruct((1, ACT), jnp.float32),
    )(part, fc_W, fc_b.reshape(1, EMB), out_W, out_b.reshape(1, ACT))

    return q_values


# speedup vs baseline: 3.8676x; 1.1263x over previous
import functools

import jax
import jax.numpy as jnp
from jax import lax
from jax.experimental import pallas as pl
from jax.experimental.pallas import tpu as pltpu
from jax.experimental.pallas import tpu_sc as plsc

N = 10000
N_PAD = 10240
E = 320000
F_IN = 128
H1 = 128
EMB = 64
ACT = 18

NUM_CORES = 2
NUM_SUBCORES = 16
NW = NUM_CORES * NUM_SUBCORES
CHUNK = 128
SCHUNKS = 80
TCHUNKS = SCHUNKS + 1
IDX_PAD = NW * TCHUNKS * CHUNK
ROWS_PER_SUB = N_PAD // NUM_SUBCORES

BLK = 1280
GRID = N_PAD // BLK


def _sc_mesh():
    return plsc.VectorSubcoreMesh(
        core_axis_name="c", subcore_axis_name="s",
        num_cores=NUM_CORES, num_subcores=NUM_SUBCORES)


def _make_deg_kernel():
    @functools.partial(
        pl.kernel,
        out_type=jax.ShapeDtypeStruct((NUM_CORES, N_PAD, 16), jnp.float32),
        mesh=_sc_mesh(),
        scratch_types=[
            pltpu.VMEM((TCHUNKS, CHUNK), jnp.int32),
            pltpu.VMEM((CHUNK, 16), jnp.float32),
            pltpu.VMEM_SHARED((N_PAD, 16), jnp.float32),
        ],
        compiler_params=pltpu.CompilerParams(use_tc_tiling_on_sc=False),
    )
    def deg_kernel(dst_hbm, ones_hbm, zeros_hbm, out_hbm, dst_v, ones_v, acc):
        c = lax.axis_index("c")
        s = lax.axis_index("s")
        wid = s * NUM_CORES + c
        row0 = s * ROWS_PER_SUB
        pltpu.sync_copy(zeros_hbm.at[pl.ds(row0, ROWS_PER_SUB)],
                        acc.at[pl.ds(row0, ROWS_PER_SUB)])
        pltpu.sync_copy(dst_hbm.at[wid], dst_v)
        pltpu.sync_copy(ones_hbm, ones_v)
        plsc.subcore_barrier()

        def body(j, carry):
            pltpu.sync_copy(ones_v, acc.at[dst_v.at[j]], add=True)
            return carry

        lax.fori_loop(0, SCHUNKS, body, 0)
        plsc.subcore_barrier()
        pltpu.sync_copy(acc.at[pl.ds(row0, ROWS_PER_SUB)],
                        out_hbm.at[c].at[pl.ds(row0, ROWS_PER_SUB)])

    return deg_kernel


def _make_edge_kernel(d, dtype=jnp.bfloat16):
    @functools.partial(
        pl.kernel,
        out_type=jax.ShapeDtypeStruct((NUM_CORES, N_PAD, d), dtype),
        mesh=_sc_mesh(),
        scratch_types=[
            pltpu.VMEM((TCHUNKS, CHUNK), jnp.int32),
            pltpu.VMEM((TCHUNKS, CHUNK), jnp.int32),
            pltpu.VMEM((CHUNK, d), dtype),
            pltpu.VMEM((CHUNK, d), dtype),
            pltpu.VMEM_SHARED((N_PAD, d), dtype),
            pltpu.VMEM_SHARED((N_PAD, d), dtype),
            pltpu.SemaphoreType.DMA,
            pltpu.SemaphoreType.DMA,
        ],
        compiler_params=pltpu.CompilerParams(use_tc_tiling_on_sc=False),
    )
    def edge_kernel(table_hbm, src_hbm, dst_hbm, zeros_hbm, out_hbm,
                    src_v, dst_v, rows0, rows1, acc, table_s, sem0, sem1):
        c = lax.axis_index("c")
        s = lax.axis_index("s")
        wid = s * NUM_CORES + c
        row0 = s * ROWS_PER_SUB

        pltpu.sync_copy(table_hbm.at[pl.ds(row0, ROWS_PER_SUB)],
                        table_s.at[pl.ds(row0, ROWS_PER_SUB)])

        @pl.when(c == 0)
        def _():
            pltpu.sync_copy(table_hbm.at[pl.ds(row0, ROWS_PER_SUB)],
                            acc.at[pl.ds(row0, ROWS_PER_SUB)])

        @pl.when(c != 0)
        def _():
            pltpu.sync_copy(zeros_hbm.at[pl.ds(row0, ROWS_PER_SUB)],
                            acc.at[pl.ds(row0, ROWS_PER_SUB)])

        pltpu.sync_copy(src_hbm.at[wid], src_v)
        pltpu.sync_copy(dst_hbm.at[wid], dst_v)
        plsc.subcore_barrier()
        pltpu.async_copy(table_s.at[src_v.at[0]], rows0, sem0).wait()

        def body(i, carry):
            c0 = 2 * i
            c1 = 2 * i + 1
            d1 = pltpu.async_copy(table_s.at[src_v.at[c1]], rows1, sem1)
            pltpu.sync_copy(rows0, acc.at[dst_v.at[c0]], add=True)
            d1.wait()
            d0 = pltpu.async_copy(table_s.at[src_v.at[c0 + 2]], rows0, sem0)
            pltpu.sync_copy(rows1, acc.at[dst_v.at[c1]], add=True)
            d0.wait()
            return carry

        lax.fori_loop(0, SCHUNKS // 2, body, 0)
        plsc.subcore_barrier()
        pltpu.sync_copy(acc.at[pl.ds(row0, ROWS_PER_SUB)],
                        out_hbm.at[c].at[pl.ds(row0, ROWS_PER_SUB)])

    return edge_kernel


def _dinv_block(deg0, deg1):
    d = deg0[:, 0:1] + deg1[:, 0:1] + 1.0
    return lax.rsqrt(d)


def _mm1_body(x_ref, w_ref, deg0_ref, deg1_ref, o_ref):
    dinv = _dinv_block(deg0_ref[...], deg1_ref[...])
    u = jnp.dot(x_ref[...], w_ref[...], preferred_element_type=jnp.float32)
    o_ref[...] = (u * dinv).astype(o_ref.dtype)


def _combine_mm2_body(p0_ref, p1_ref, deg0_ref, deg1_ref, b1_ref, w2_ref, o_ref):
    dinv = _dinv_block(deg0_ref[...], deg1_ref[...])
    psum = p0_ref[...].astype(jnp.float32) + p1_ref[...].astype(jnp.float32)
    x1 = jnp.maximum(psum * dinv + b1_ref[...], 0.0)
    u = jnp.dot(x1, w2_ref[...], preferred_element_type=jnp.float32)
    o_ref[...] = (u * dinv).astype(o_ref.dtype)


def _pool_body(q0_ref, q1_ref, deg0_ref, deg1_ref, b2_ref, o_ref):
    i = pl.program_id(0)
    dinv = _dinv_block(deg0_ref[...], deg1_ref[...])
    qsum = q0_ref[...].astype(jnp.float32) + q1_ref[...].astype(jnp.float32)
    x2 = jnp.maximum(qsum * dinv + b2_ref[...], 0.0)
    rid = i * BLK + lax.broadcasted_iota(jnp.int32, (BLK, EMB), 0)
    x2 = jnp.where(rid < N, x2, 0.0)
    o_ref[...] = jnp.sum(x2, axis=0, keepdims=True).reshape(1, 1, EMB)


def _head_body(s_ref, fw_ref, fb_ref, ow_ref, ob_ref, o_ref):
    pooled = jnp.sum(s_ref[...], axis=0, keepdims=True) * (1.0 / N)
    hid = jnp.dot(pooled, fw_ref[...], preferred_element_type=jnp.float32)
    hid = jnp.maximum(hid + fb_ref[...], 0.0)
    o_ref[...] = jnp.dot(hid, ow_ref[...], preferred_element_type=jnp.float32) + ob_ref[...]


def _row_spec(width):
    return pl.BlockSpec((BLK, width), lambda i: (i, 0))


def _full_spec(shape):
    return pl.BlockSpec(shape, lambda i: (0,) * len(shape))


def kernel(node_features, edge_index, W1, b1, W2, b2, fc_W, fc_b, out_W, out_b):
    x = node_features.reshape(-1, F_IN)
    ei = edge_index.reshape(2, -1).astype(jnp.int32)
    src = ei[0]
    dst = ei[1]

    e_sc = NW * SCHUNKS * CHUNK
    pad_idx = jnp.full((e_sc - E,), N, jnp.int32)
    tail = jnp.full((NW, TCHUNKS - SCHUNKS, CHUNK), N, jnp.int32)
    src_p = jnp.concatenate(
        [jnp.concatenate([src, pad_idx]).reshape(NW, SCHUNKS, CHUNK), tail], axis=1)
    dst_p = jnp.concatenate(
        [jnp.concatenate([dst, pad_idx]).reshape(NW, SCHUNKS, CHUNK), tail], axis=1)

    x_pad = jnp.pad(x, ((0, N_PAD - N), (0, 0)))
    zeros16 = jnp.zeros((N_PAD, 16), jnp.float32)
    zeros128 = jnp.zeros((N_PAD, H1), jnp.bfloat16)
    zeros64 = jnp.zeros((N_PAD, EMB), jnp.bfloat16)
    ones16 = jnp.ones((CHUNK, 16), jnp.float32)

    deg = _make_deg_kernel()(dst_p, ones16, zeros16)
    deg0, deg1 = deg[0], deg[1]

    h1p = pl.pallas_call(
        _mm1_body,
        grid=(GRID,),
        in_specs=[_row_spec(F_IN), _full_spec((F_IN, H1)),
                  _row_spec(16), _row_spec(16)],
        out_specs=_row_spec(H1),
        out_shape=jax.ShapeDtypeStruct((N_PAD, H1), jnp.bfloat16),
    )(x_pad, W1, deg0, deg1)

    p = _make_edge_kernel(H1)(h1p, src_p, dst_p, zeros128)

    h2p = pl.pallas_call(
        _combine_mm2_body,
        grid=(GRID,),
        in_specs=[_row_spec(H1), _row_spec(H1), _row_spec(16), _row_spec(16),
                  _full_spec((1, H1)), _full_spec((H1, EMB))],
        out_specs=_row_spec(EMB),
        out_shape=jax.ShapeDtypeStruct((N_PAD, EMB), jnp.bfloat16),
    )(p[0], p[1], deg0, deg1, b1.reshape(1, H1), W2)

    q = _make_edge_kernel(EMB)(h2p, src_p, dst_p, zeros64)

    part = pl.pallas_call(
        _pool_body,
        grid=(GRID,),
        in_specs=[_row_spec(EMB), _row_spec(EMB), _row_spec(16), _row_spec(16),
                  _full_spec((1, EMB))],
        out_specs=pl.BlockSpec((1, 1, EMB), lambda i: (i, 0, 0)),
        out_shape=jax.ShapeDtypeStruct((GRID, 1, EMB), jnp.float32),
    )(q[0], q[1], deg0, deg1, b2.reshape(1, EMB))
    part = part.reshape(GRID, EMB)

    q_values = pl.pallas_call(
        _head_body,
        in_specs=[pl.BlockSpec((GRID, EMB), lambda: (0, 0)),
                  pl.BlockSpec((EMB, EMB), lambda: (0, 0)),
                  pl.BlockSpec((1, EMB), lambda: (0, 0)),
                  pl.BlockSpec((EMB, ACT), lambda: (0, 0)),
                  pl.BlockSpec((1, ACT), lambda: (0, 0))],
        out_specs=pl.BlockSpec((1, ACT), lambda: (0, 0)),
        out_shape=jax.ShapeDtypeStruct((1, ACT), jnp.float32),
    )(part, fc_W, fc_b.reshape(1, EMB), out_W, out_b.reshape(1, ACT))

    return q_values


# speedup vs baseline: 3.9040x; 1.0094x over previous
import functools

import jax
import jax.numpy as jnp
from jax import lax
from jax.experimental import pallas as pl
from jax.experimental.pallas import tpu as pltpu
from jax.experimental.pallas import tpu_sc as plsc

N = 10000
N_PAD = 10240
E = 320000
F_IN = 128
H1 = 128
EMB = 64
ACT = 18

NUM_CORES = 2
NUM_SUBCORES = 16
NW = NUM_CORES * NUM_SUBCORES
CHUNK = 128
SCHUNKS = 80
TCHUNKS = SCHUNKS
IDX_PAD = NW * TCHUNKS * CHUNK
ROWS_PER_SUB = N_PAD // NUM_SUBCORES

BLK = 1280
GRID = N_PAD // BLK


def _sc_mesh():
    return plsc.VectorSubcoreMesh(
        core_axis_name="c", subcore_axis_name="s",
        num_cores=NUM_CORES, num_subcores=NUM_SUBCORES)


def _make_deg_kernel():
    @functools.partial(
        pl.kernel,
        out_type=jax.ShapeDtypeStruct((NUM_CORES, N_PAD, 16), jnp.float32),
        mesh=_sc_mesh(),
        scratch_types=[
            pltpu.VMEM((TCHUNKS, CHUNK), jnp.int32),
            pltpu.VMEM((CHUNK, 16), jnp.float32),
            pltpu.VMEM_SHARED((N_PAD, 16), jnp.float32),
        ],
        compiler_params=pltpu.CompilerParams(use_tc_tiling_on_sc=False),
    )
    def deg_kernel(dst_hbm, ones_hbm, zeros_hbm, out_hbm, dst_v, ones_v, acc):
        c = lax.axis_index("c")
        s = lax.axis_index("s")
        wid = s * NUM_CORES + c
        row0 = s * ROWS_PER_SUB
        pltpu.sync_copy(zeros_hbm.at[pl.ds(row0, ROWS_PER_SUB)],
                        acc.at[pl.ds(row0, ROWS_PER_SUB)])
        pltpu.sync_copy(dst_hbm.at[wid], dst_v)
        pltpu.sync_copy(ones_hbm, ones_v)
        plsc.subcore_barrier()

        def body(j, carry):
            pltpu.sync_copy(ones_v, acc.at[dst_v.at[j]], add=True)
            return carry

        lax.fori_loop(0, SCHUNKS, body, 0)
        plsc.subcore_barrier()
        pltpu.sync_copy(acc.at[pl.ds(row0, ROWS_PER_SUB)],
                        out_hbm.at[c].at[pl.ds(row0, ROWS_PER_SUB)])

    return deg_kernel


def _make_edge_kernel(d, dtype=jnp.bfloat16):
    @functools.partial(
        pl.kernel,
        out_type=jax.ShapeDtypeStruct((NUM_CORES, N_PAD, d), dtype),
        mesh=_sc_mesh(),
        scratch_types=[
            pltpu.VMEM((TCHUNKS, CHUNK), jnp.int32),
            pltpu.VMEM((TCHUNKS, CHUNK), jnp.int32),
            pltpu.VMEM((CHUNK, d), dtype),
            pltpu.VMEM((CHUNK, d), dtype),
            pltpu.VMEM_SHARED((N_PAD, d), dtype),
            pltpu.VMEM_SHARED((N_PAD, d), dtype),
            pltpu.SemaphoreType.DMA,
            pltpu.SemaphoreType.DMA,
        ],
        compiler_params=pltpu.CompilerParams(use_tc_tiling_on_sc=False),
    )
    def edge_kernel(table_hbm, src_hbm, dst_hbm, zeros_hbm, out_hbm,
                    src_v, dst_v, rows0, rows1, acc, table_s, sem0, sem1):
        c = lax.axis_index("c")
        s = lax.axis_index("s")
        wid = s * NUM_CORES + c
        row0 = s * ROWS_PER_SUB

        pltpu.sync_copy(table_hbm.at[pl.ds(row0, ROWS_PER_SUB)],
                        table_s.at[pl.ds(row0, ROWS_PER_SUB)])

        @pl.when(c == 0)
        def _():
            pltpu.sync_copy(table_hbm.at[pl.ds(row0, ROWS_PER_SUB)],
                            acc.at[pl.ds(row0, ROWS_PER_SUB)])

        @pl.when(c != 0)
        def _():
            pltpu.sync_copy(zeros_hbm.at[pl.ds(row0, ROWS_PER_SUB)],
                            acc.at[pl.ds(row0, ROWS_PER_SUB)])

        pltpu.sync_copy(src_hbm.at[wid], src_v)
        pltpu.sync_copy(dst_hbm.at[wid], dst_v)
        plsc.subcore_barrier()
        pltpu.async_copy(table_s.at[src_v.at[0]], rows0, sem0).wait()

        def body(i, carry):
            c0 = 2 * i
            c1 = 2 * i + 1
            d1 = pltpu.async_copy(table_s.at[src_v.at[c1]], rows1, sem1)
            pltpu.sync_copy(rows0, acc.at[dst_v.at[c0]], add=True)
            d1.wait()
            d0 = pltpu.async_copy(table_s.at[src_v.at[c0 + 2]], rows0, sem0)
            pltpu.sync_copy(rows1, acc.at[dst_v.at[c1]], add=True)
            d0.wait()
            return carry

        lax.fori_loop(0, SCHUNKS // 2 - 1, body, 0)
        dl = pltpu.async_copy(table_s.at[src_v.at[SCHUNKS - 1]], rows1, sem1)
        pltpu.sync_copy(rows0, acc.at[dst_v.at[SCHUNKS - 2]], add=True)
        dl.wait()
        pltpu.sync_copy(rows1, acc.at[dst_v.at[SCHUNKS - 1]], add=True)
        plsc.subcore_barrier()
        pltpu.sync_copy(acc.at[pl.ds(row0, ROWS_PER_SUB)],
                        out_hbm.at[c].at[pl.ds(row0, ROWS_PER_SUB)])

    return edge_kernel


def _dinv_block(deg0, deg1):
    d = deg0[:, 0:1] + deg1[:, 0:1] + 1.0
    return lax.rsqrt(d)


def _mm1_body(x_ref, w_ref, deg0_ref, deg1_ref, o_ref):
    dinv = _dinv_block(deg0_ref[...], deg1_ref[...])
    u = jnp.dot(x_ref[...], w_ref[...], preferred_element_type=jnp.float32)
    o_ref[...] = (u * dinv).astype(o_ref.dtype)


def _combine_mm2_body(p0_ref, p1_ref, deg0_ref, deg1_ref, b1_ref, w2_ref, o_ref):
    dinv = _dinv_block(deg0_ref[...], deg1_ref[...])
    psum = p0_ref[...].astype(jnp.float32) + p1_ref[...].astype(jnp.float32)
    x1 = jnp.maximum(psum * dinv + b1_ref[...], 0.0)
    u = jnp.dot(x1, w2_ref[...], preferred_element_type=jnp.float32)
    o_ref[...] = (u * dinv).astype(o_ref.dtype)


def _pool_head_body(q0_ref, q1_ref, deg0_ref, deg1_ref, b2_ref,
                    fw_ref, fb_ref, ow_ref, ob_ref, o_ref, acc_ref):
    i = pl.program_id(0)
    dinv = _dinv_block(deg0_ref[...], deg1_ref[...])
    qsum = q0_ref[...].astype(jnp.float32) + q1_ref[...].astype(jnp.float32)
    x2 = jnp.maximum(qsum * dinv + b2_ref[...], 0.0)
    rid = i * BLK + lax.broadcasted_iota(jnp.int32, (BLK, EMB), 0)
    x2 = jnp.where(rid < N, x2, 0.0)
    s = jnp.sum(x2, axis=0, keepdims=True)

    @pl.when(i == 0)
    def _():
        acc_ref[...] = s

    @pl.when(i > 0)
    def _():
        acc_ref[...] += s

    @pl.when(i == GRID - 1)
    def _():
        pooled = acc_ref[...] * (1.0 / N)
        hid = jnp.dot(pooled, fw_ref[...], preferred_element_type=jnp.float32)
        hid = jnp.maximum(hid + fb_ref[...], 0.0)
        o_ref[...] = (jnp.dot(hid, ow_ref[...], preferred_element_type=jnp.float32)
                      + ob_ref[...])


def _row_spec(width):
    return pl.BlockSpec((BLK, width), lambda i: (i, 0))


def _full_spec(shape):
    return pl.BlockSpec(shape, lambda i: (0,) * len(shape))


def kernel(node_features, edge_index, W1, b1, W2, b2, fc_W, fc_b, out_W, out_b):
    x = node_features.reshape(-1, F_IN)
    ei = edge_index.reshape(2, -1).astype(jnp.int32)
    src = ei[0]
    dst = ei[1]

    e_sc = NW * SCHUNKS * CHUNK
    pad_idx = jnp.full((e_sc - E,), N, jnp.int32)
    src_p = jnp.concatenate([src, pad_idx]).reshape(NW, SCHUNKS, CHUNK)
    dst_p = jnp.concatenate([dst, pad_idx]).reshape(NW, SCHUNKS, CHUNK)

    x_pad = jnp.pad(x, ((0, N_PAD - N), (0, 0)))
    zeros16 = jnp.zeros((N_PAD, 16), jnp.float32)
    zeros128 = jnp.zeros((N_PAD, H1), jnp.bfloat16)
    zeros64 = jnp.zeros((N_PAD, EMB), jnp.bfloat16)
    ones16 = jnp.ones((CHUNK, 16), jnp.float32)

    deg = _make_deg_kernel()(dst_p, ones16, zeros16)
    deg0, deg1 = deg[0], deg[1]

    h1p = pl.pallas_call(
        _mm1_body,
        grid=(GRID,),
        in_specs=[_row_spec(F_IN), _full_spec((F_IN, H1)),
                  _row_spec(16), _row_spec(16)],
        out_specs=_row_spec(H1),
        out_shape=jax.ShapeDtypeStruct((N_PAD, H1), jnp.bfloat16),
    )(x_pad, W1, deg0, deg1)

    p = _make_edge_kernel(H1)(h1p, src_p, dst_p, zeros128)

    h2p = pl.pallas_call(
        _combine_mm2_body,
        grid=(GRID,),
        in_specs=[_row_spec(H1), _row_spec(H1), _row_spec(16), _row_spec(16),
                  _full_spec((1, H1)), _full_spec((H1, EMB))],
        out_specs=_row_spec(EMB),
        out_shape=jax.ShapeDtypeStruct((N_PAD, EMB), jnp.bfloat16),
    )(p[0], p[1], deg0, deg1, b1.reshape(1, H1), W2)

    q = _make_edge_kernel(EMB)(h2p, src_p, dst_p, zeros64)

    q_values = pl.pallas_call(
        _pool_head_body,
        grid=(GRID,),
        in_specs=[_row_spec(EMB), _row_spec(EMB), _row_spec(16), _row_spec(16),
                  _full_spec((1, EMB)), _full_spec((EMB, EMB)),
                  _full_spec((1, EMB)), _full_spec((EMB, ACT)),
                  _full_spec((1, ACT))],
        out_specs=pl.BlockSpec((1, ACT), lambda i: (0, 0)),
        out_shape=jax.ShapeDtypeStruct((1, ACT), jnp.float32),
        scratch_shapes=[pltpu.VMEM((1, EMB), jnp.float32)],
    )(q[0], q[1], deg0, deg1, b2.reshape(1, EMB), fc_W,
      fc_b.reshape(1, EMB), out_W, out_b.reshape(1, ACT))

    return q_values


# speedup vs baseline: 3.9048x; 1.0002x over previous
import functools

import jax
import jax.numpy as jnp
from jax import lax
from jax.experimental import pallas as pl
from jax.experimental.pallas import tpu as pltpu
from jax.experimental.pallas import tpu_sc as plsc

N = 10000
N_PAD = 10240
E = 320000
F_IN = 128
H1 = 128
EMB = 64
ACT = 18

NUM_CORES = 2
NUM_SUBCORES = 16
NW = NUM_CORES * NUM_SUBCORES
CHUNK = 128
SCHUNKS = 80
TCHUNKS = SCHUNKS
IDX_PAD = NW * TCHUNKS * CHUNK
ROWS_PER_SUB = N_PAD // NUM_SUBCORES

BLK = 1280
GRID = N_PAD // BLK


def _sc_mesh():
    return plsc.VectorSubcoreMesh(
        core_axis_name="c", subcore_axis_name="s",
        num_cores=NUM_CORES, num_subcores=NUM_SUBCORES)


def _make_deg_kernel():
    @functools.partial(
        pl.kernel,
        out_type=jax.ShapeDtypeStruct((NUM_CORES, N_PAD, 16), jnp.float32),
        mesh=_sc_mesh(),
        scratch_types=[
            pltpu.VMEM((TCHUNKS, CHUNK), jnp.int32),
            pltpu.VMEM((CHUNK, 16), jnp.float32),
            pltpu.VMEM_SHARED((N_PAD, 16), jnp.float32),
            pltpu.SemaphoreType.DMA,
            pltpu.SemaphoreType.DMA,
        ],
        compiler_params=pltpu.CompilerParams(use_tc_tiling_on_sc=False),
    )
    def deg_kernel(dst_hbm, ones_hbm, zeros_hbm, out_hbm, dst_v, ones_v, acc,
                   sem0, sem1):
        c = lax.axis_index("c")
        s = lax.axis_index("s")
        wid = s * NUM_CORES + c
        row0 = s * ROWS_PER_SUB
        pltpu.sync_copy(zeros_hbm.at[pl.ds(row0, ROWS_PER_SUB)],
                        acc.at[pl.ds(row0, ROWS_PER_SUB)])
        pltpu.sync_copy(dst_hbm.at[wid], dst_v)
        pltpu.sync_copy(ones_hbm, ones_v)
        plsc.subcore_barrier()

        def body(i, carry):
            d0 = pltpu.async_copy(ones_v, acc.at[dst_v.at[2 * i]], sem0,
                                  add=True)
            d1 = pltpu.async_copy(ones_v, acc.at[dst_v.at[2 * i + 1]], sem1,
                                  add=True)
            d0.wait()
            d1.wait()
            return carry

        lax.fori_loop(0, SCHUNKS // 2, body, 0)
        plsc.subcore_barrier()
        pltpu.sync_copy(acc.at[pl.ds(row0, ROWS_PER_SUB)],
                        out_hbm.at[c].at[pl.ds(row0, ROWS_PER_SUB)])

    return deg_kernel


def _make_edge_kernel(d, dtype=jnp.bfloat16):
    @functools.partial(
        pl.kernel,
        out_type=jax.ShapeDtypeStruct((NUM_CORES, N_PAD, d), dtype),
        mesh=_sc_mesh(),
        scratch_types=[
            pltpu.VMEM((TCHUNKS, CHUNK), jnp.int32),
            pltpu.VMEM((TCHUNKS, CHUNK), jnp.int32),
            pltpu.VMEM((CHUNK, d), dtype),
            pltpu.VMEM((CHUNK, d), dtype),
            pltpu.VMEM_SHARED((N_PAD, d), dtype),
            pltpu.VMEM_SHARED((N_PAD, d), dtype),
            pltpu.SemaphoreType.DMA,
            pltpu.SemaphoreType.DMA,
        ],
        compiler_params=pltpu.CompilerParams(use_tc_tiling_on_sc=False),
    )
    def edge_kernel(table_hbm, src_hbm, dst_hbm, zeros_hbm, out_hbm,
                    src_v, dst_v, rows0, rows1, acc, table_s, sem0, sem1):
        c = lax.axis_index("c")
        s = lax.axis_index("s")
        wid = s * NUM_CORES + c
        row0 = s * ROWS_PER_SUB

        pltpu.sync_copy(table_hbm.at[pl.ds(row0, ROWS_PER_SUB)],
                        table_s.at[pl.ds(row0, ROWS_PER_SUB)])

        @pl.when(c == 0)
        def _():
            pltpu.sync_copy(table_hbm.at[pl.ds(row0, ROWS_PER_SUB)],
                            acc.at[pl.ds(row0, ROWS_PER_SUB)])

        @pl.when(c != 0)
        def _():
            pltpu.sync_copy(zeros_hbm.at[pl.ds(row0, ROWS_PER_SUB)],
                            acc.at[pl.ds(row0, ROWS_PER_SUB)])

        pltpu.sync_copy(src_hbm.at[wid], src_v)
        pltpu.sync_copy(dst_hbm.at[wid], dst_v)
        plsc.subcore_barrier()
        pltpu.async_copy(table_s.at[src_v.at[0]], rows0, sem0).wait()

        def body(i, carry):
            c0 = 2 * i
            c1 = 2 * i + 1
            d1 = pltpu.async_copy(table_s.at[src_v.at[c1]], rows1, sem1)
            pltpu.sync_copy(rows0, acc.at[dst_v.at[c0]], add=True)
            d1.wait()
            d0 = pltpu.async_copy(table_s.at[src_v.at[c0 + 2]], rows0, sem0)
            pltpu.sync_copy(rows1, acc.at[dst_v.at[c1]], add=True)
            d0.wait()
            return carry

        lax.fori_loop(0, SCHUNKS // 2 - 1, body, 0)
        dl = pltpu.async_copy(table_s.at[src_v.at[SCHUNKS - 1]], rows1, sem1)
        pltpu.sync_copy(rows0, acc.at[dst_v.at[SCHUNKS - 2]], add=True)
        dl.wait()
        pltpu.sync_copy(rows1, acc.at[dst_v.at[SCHUNKS - 1]], add=True)
        plsc.subcore_barrier()
        pltpu.sync_copy(acc.at[pl.ds(row0, ROWS_PER_SUB)],
                        out_hbm.at[c].at[pl.ds(row0, ROWS_PER_SUB)])

    return edge_kernel


def _dinv_block(deg0, deg1):
    d = deg0[:, 0:1] + deg1[:, 0:1] + 1.0
    return lax.rsqrt(d)


def _mm1_body(x_ref, w_ref, deg0_ref, deg1_ref, o_ref):
    dinv = _dinv_block(deg0_ref[...], deg1_ref[...])
    u = jnp.dot(x_ref[...], w_ref[...], preferred_element_type=jnp.float32)
    o_ref[...] = (u * dinv).astype(o_ref.dtype)


def _combine_mm2_body(p0_ref, p1_ref, deg0_ref, deg1_ref, b1_ref, w2_ref, o_ref):
    dinv = _dinv_block(deg0_ref[...], deg1_ref[...])
    psum = p0_ref[...].astype(jnp.float32) + p1_ref[...].astype(jnp.float32)
    x1 = jnp.maximum(psum * dinv + b1_ref[...], 0.0)
    u = jnp.dot(x1, w2_ref[...], preferred_element_type=jnp.float32)
    o_ref[...] = (u * dinv).astype(o_ref.dtype)


def _pool_head_body(q0_ref, q1_ref, deg0_ref, deg1_ref, b2_ref,
                    fw_ref, fb_ref, ow_ref, ob_ref, o_ref, acc_ref):
    i = pl.program_id(0)
    dinv = _dinv_block(deg0_ref[...], deg1_ref[...])
    qsum = q0_ref[...].astype(jnp.float32) + q1_ref[...].astype(jnp.float32)
    x2 = jnp.maximum(qsum * dinv + b2_ref[...], 0.0)
    rid = i * BLK + lax.broadcasted_iota(jnp.int32, (BLK, EMB), 0)
    x2 = jnp.where(rid < N, x2, 0.0)
    s = jnp.sum(x2, axis=0, keepdims=True)

    @pl.when(i == 0)
    def _():
        acc_ref[...] = s

    @pl.when(i > 0)
    def _():
        acc_ref[...] += s

    @pl.when(i == GRID - 1)
    def _():
        pooled = acc_ref[...] * (1.0 / N)
        hid = jnp.dot(pooled, fw_ref[...], preferred_element_type=jnp.float32)
        hid = jnp.maximum(hid + fb_ref[...], 0.0)
        o_ref[...] = (jnp.dot(hid, ow_ref[...], preferred_element_type=jnp.float32)
                      + ob_ref[...])


def _row_spec(width):
    return pl.BlockSpec((BLK, width), lambda i: (i, 0))


def _full_spec(shape):
    return pl.BlockSpec(shape, lambda i: (0,) * len(shape))


def kernel(node_features, edge_index, W1, b1, W2, b2, fc_W, fc_b, out_W, out_b):
    x = node_features.reshape(-1, F_IN)
    ei = edge_index.reshape(2, -1).astype(jnp.int32)
    src = ei[0]
    dst = ei[1]

    e_sc = NW * SCHUNKS * CHUNK
    src_p = jnp.pad(src, (0, e_sc - E), constant_values=N).reshape(
        NW, SCHUNKS, CHUNK)
    dst_p = jnp.pad(dst, (0, e_sc - E), constant_values=N).reshape(
        NW, SCHUNKS, CHUNK)

    x_pad = jnp.pad(x, ((0, N_PAD - N), (0, 0)))
    zeros16 = jnp.zeros((N_PAD, 16), jnp.float32)
    zeros128 = jnp.zeros((N_PAD, H1), jnp.bfloat16)
    zeros64 = jnp.zeros((N_PAD, EMB), jnp.bfloat16)
    ones16 = jnp.ones((CHUNK, 16), jnp.float32)

    deg = _make_deg_kernel()(dst_p, ones16, zeros16)
    deg0, deg1 = deg[0], deg[1]

    h1p = pl.pallas_call(
        _mm1_body,
        grid=(GRID,),
        in_specs=[_row_spec(F_IN), _full_spec((F_IN, H1)),
                  _row_spec(16), _row_spec(16)],
        out_specs=_row_spec(H1),
        out_shape=jax.ShapeDtypeStruct((N_PAD, H1), jnp.bfloat16),
    )(x_pad, W1, deg0, deg1)

    p = _make_edge_kernel(H1)(h1p, src_p, dst_p, zeros128)

    h2p = pl.pallas_call(
        _combine_mm2_body,
        grid=(GRID,),
        in_specs=[_row_spec(H1), _row_spec(H1), _row_spec(16), _row_spec(16),
                  _full_spec((1, H1)), _full_spec((H1, EMB))],
        out_specs=_row_spec(EMB),
        out_shape=jax.ShapeDtypeStruct((N_PAD, EMB), jnp.bfloat16),
    )(p[0], p[1], deg0, deg1, b1.reshape(1, H1), W2)

    q = _make_edge_kernel(EMB)(h2p, src_p, dst_p, zeros64)

    q_values = pl.pallas_call(
        _pool_head_body,
        grid=(GRID,),
        in_specs=[_row_spec(EMB), _row_spec(EMB), _row_spec(16), _row_spec(16),
                  _full_spec((1, EMB)), _full_spec((EMB, EMB)),
                  _full_spec((1, EMB)), _full_spec((EMB, ACT)),
                  _full_spec((1, ACT))],
        out_specs=pl.BlockSpec((1, ACT), lambda i: (0, 0)),
        out_shape=jax.ShapeDtypeStruct((1, ACT), jnp.float32),
        scratch_shapes=[pltpu.VMEM((1, EMB), jnp.float32)],
    )(q[0], q[1], deg0, deg1, b2.reshape(1, EMB), fc_W,
      fc_b.reshape(1, EMB), out_W, out_b.reshape(1, ACT))

    return q_values
